# baseline (speedup 1.0000x reference)
import functools

import jax
import jax.numpy as jnp
from jax.experimental import pallas as pl
from jax.experimental.pallas import tpu as pltpu

_TEMPERATURE = 10.0
_BM = 8192


def _flash_body(addr_ref, keys_ref, values_ref, out_ref, m_ref, acc_ref,
                *, num_blocks):
    i = pl.program_id(0)

    @pl.when(i == 0)
    def _init():
        m_ref[...] = jnp.full_like(m_ref, -jnp.inf)
        acc_ref[...] = jnp.zeros_like(acc_ref)

    addr = (addr_ref[...] * _TEMPERATURE).astype(jnp.bfloat16)

    m = m_ref[...]
    acc = acc_ref[...]
    half = _BM // 2
    for h in range(2):
        keys = keys_ref[pl.ds(h * half, half), :].astype(jnp.bfloat16)
        vals = values_ref[pl.ds(h * half, half), :].astype(jnp.bfloat16)
        ones = jnp.ones((half, 1), jnp.bfloat16)
        vals1 = jnp.concatenate([vals, ones], axis=1)

        s = jax.lax.dot_general(
            addr, keys,
            (((1,), (1,)), ((), ())),
            preferred_element_type=jnp.float32,
        ).astype(jnp.bfloat16)
        m_new = jnp.maximum(m, jnp.max(s, axis=1, keepdims=True))
        alpha = jnp.exp(m - m_new)
        p = jnp.exp(s - m_new)
        pv = jax.lax.dot_general(
            p, vals1,
            (((1,), (0,)), ((), ())),
            preferred_element_type=jnp.float32,
        )
        acc = acc * alpha.astype(jnp.float32) + pv
        m = m_new
    m_ref[...] = m
    acc_ref[...] = acc

    @pl.when(i == num_blocks - 1)
    def _fini():
        acc = acc_ref[...]
        out_ref[...] = acc[:, :-1] / acc[:, -1:]


@jax.jit
def kernel(keys, values, addr):
    M, D = keys.shape
    Q = addr.shape[0]
    V = values.shape[1]
    num_blocks = M // _BM

    return pl.pallas_call(
        functools.partial(_flash_body, num_blocks=num_blocks),
        grid=(num_blocks,),
        in_specs=[
            pl.BlockSpec((Q, D), lambda i: (0, 0)),
            pl.BlockSpec((_BM, D), lambda i: (i, 0)),
            pl.BlockSpec((_BM, V), lambda i: (i, 0)),
        ],
        out_specs=pl.BlockSpec((Q, V), lambda i: (0, 0)),
        out_shape=jax.ShapeDtypeStruct((Q, V), jnp.float32),
        scratch_shapes=[
            pltpu.VMEM((Q, 1), jnp.bfloat16),
            pltpu.VMEM((Q, V + 1), jnp.float32),
        ],
        compiler_params=pltpu.CompilerParams(
            dimension_semantics=("arbitrary",),
        ),
    )(addr, keys, values)


# speedup vs baseline: 1.0438x; 1.0438x over previous
import functools

import jax
import jax.numpy as jnp
from jax.experimental import pallas as pl
from jax.experimental.pallas import tpu as pltpu

_TEMPERATURE = 10.0
_BM = 8192
_NQ = 2


def _flash_body(addr_ref, keys_ref, values_ref, out_ref, m_ref, acc_ref,
                *, num_blocks):
    i = pl.program_id(1)

    @pl.when(i == 0)
    def _init():
        m_ref[...] = jnp.full_like(m_ref, -jnp.inf)
        acc_ref[...] = jnp.zeros_like(acc_ref)

    addr = (addr_ref[...] * _TEMPERATURE).astype(jnp.bfloat16)
    keys = keys_ref[...].astype(jnp.bfloat16)
    vals = values_ref[...].astype(jnp.bfloat16)
    ones = jnp.ones((vals.shape[0], 1), jnp.bfloat16)
    vals1 = jnp.concatenate([vals, ones], axis=1)

    s = jax.lax.dot_general(
        addr, keys,
        (((1,), (1,)), ((), ())),
        preferred_element_type=jnp.float32,
    ).astype(jnp.bfloat16)
    m_prev = m_ref[...]
    m_new = jnp.maximum(m_prev, jnp.max(s, axis=1, keepdims=True))
    alpha = jnp.exp(m_prev - m_new)
    p = jnp.exp(s - m_new)
    pv = jax.lax.dot_general(
        p, vals1,
        (((1,), (0,)), ((), ())),
        preferred_element_type=jnp.float32,
    )
    m_ref[...] = m_new
    acc_ref[...] = acc_ref[...] * alpha.astype(jnp.float32) + pv

    @pl.when(i == num_blocks - 1)
    def _fini():
        acc = acc_ref[...]
        out_ref[...] = acc[:, :-1] / acc[:, -1:]


@jax.jit
def kernel(keys, values, addr):
    M, D = keys.shape
    Q = addr.shape[0]
    V = values.shape[1]
    num_blocks = M // _BM
    qb = Q // _NQ

    return pl.pallas_call(
        functools.partial(_flash_body, num_blocks=num_blocks),
        grid=(_NQ, num_blocks),
        in_specs=[
            pl.BlockSpec((qb, D), lambda q, i: (q, 0)),
            pl.BlockSpec((_BM, D), lambda q, i: (i, 0)),
            pl.BlockSpec((_BM, V), lambda q, i: (i, 0)),
        ],
        out_specs=pl.BlockSpec((qb, V), lambda q, i: (q, 0)),
        out_shape=jax.ShapeDtypeStruct((Q, V), jnp.float32),
        scratch_shapes=[
            pltpu.VMEM((qb, 1), jnp.bfloat16),
            pltpu.VMEM((qb, V + 1), jnp.float32),
        ],
        compiler_params=pltpu.CompilerParams(
            dimension_semantics=("parallel", "arbitrary"),
        ),
    )(addr, keys, values)


# speedup vs baseline: 1.1402x; 1.0924x over previous
import functools

import jax
import jax.numpy as jnp
from jax.experimental import pallas as pl
from jax.experimental.pallas import tpu as pltpu

_TEMPERATURE = 10.0
_BM = 8192


def _flash_body(addr_ref, keysT_ref, values_ref, out_ref, m_ref, acc_ref,
                *, num_blocks):
    i = pl.program_id(0)

    @pl.when(i == 0)
    def _init():
        m_ref[...] = jnp.full_like(m_ref, -jnp.inf)
        acc_ref[...] = jnp.zeros_like(acc_ref)

    s = jax.lax.dot_general(
        addr_ref[...], keysT_ref[...],
        (((1,), (0,)), ((), ())),
        preferred_element_type=jnp.float32,
    ).astype(jnp.bfloat16)
    m_prev = m_ref[...]
    m_new = jnp.maximum(m_prev, jnp.max(s, axis=1, keepdims=True))
    alpha = jnp.exp(m_prev - m_new)
    p = jnp.exp(s - m_new)
    pv = jax.lax.dot_general(
        p, values_ref[...],
        (((1,), (0,)), ((), ())),
        preferred_element_type=jnp.float32,
    )
    m_ref[...] = m_new
    acc_ref[...] = acc_ref[...] * alpha.astype(jnp.float32) + pv

    @pl.when(i == num_blocks - 1)
    def _fini():
        acc = acc_ref[...]
        out_ref[...] = acc[:, :-1] / acc[:, -1:]


@jax.jit
def kernel(keys, values, addr):
    M, D = keys.shape
    Q = addr.shape[0]
    V = values.shape[1]
    num_blocks = M // _BM

    addr_s = (addr * _TEMPERATURE).astype(jnp.bfloat16)
    keysT = keys.astype(jnp.bfloat16).T
    vals1 = jnp.concatenate(
        [values, jnp.ones((M, 1), values.dtype)], axis=1
    ).astype(jnp.bfloat16)

    return pl.pallas_call(
        functools.partial(_flash_body, num_blocks=num_blocks),
        grid=(num_blocks,),
        in_specs=[
            pl.BlockSpec((Q, D), lambda i: (0, 0)),
            pl.BlockSpec((D, _BM), lambda i: (0, i)),
            pl.BlockSpec((_BM, V + 1), lambda i: (i, 0)),
        ],
        out_specs=pl.BlockSpec((Q, V), lambda i: (0, 0)),
        out_shape=jax.ShapeDtypeStruct((Q, V), jnp.float32),
        scratch_shapes=[
            pltpu.VMEM((Q, 1), jnp.bfloat16),
            pltpu.VMEM((Q, V + 1), jnp.float32),
        ],
        compiler_params=pltpu.CompilerParams(
            dimension_semantics=("arbitrary",),
        ),
    )(addr_s, keysT, vals1)


# speedup vs baseline: 1.1516x; 1.0100x over previous
import functools

import jax
import jax.numpy as jnp
from jax.experimental import pallas as pl
from jax.experimental.pallas import tpu as pltpu

_TEMPERATURE = 10.0
_BM = 16384


def _flash_body(addr_ref, keysT_ref, values_ref, out_ref, m_ref, acc_ref,
                *, num_blocks):
    i = pl.program_id(0)

    @pl.when(i == 0)
    def _init():
        m_ref[...] = jnp.full_like(m_ref, -jnp.inf)
        acc_ref[...] = jnp.zeros_like(acc_ref)

    s = jax.lax.dot_general(
        addr_ref[...], keysT_ref[...],
        (((1,), (0,)), ((), ())),
        preferred_element_type=jnp.float32,
    ).astype(jnp.bfloat16)
    m_prev = m_ref[...]
    m_new = jnp.maximum(m_prev, jnp.max(s, axis=1, keepdims=True))
    alpha = jnp.exp(m_prev - m_new)
    p = jnp.exp(s - m_new)
    pv = jax.lax.dot_general(
        p, values_ref[...],
        (((1,), (0,)), ((), ())),
        preferred_element_type=jnp.float32,
    )
    m_ref[...] = m_new
    acc_ref[...] = acc_ref[...] * alpha.astype(jnp.float32) + pv

    @pl.when(i == num_blocks - 1)
    def _fini():
        acc = acc_ref[...]
        out_ref[...] = acc[:, :-1] / acc[:, -1:]


@jax.jit
def kernel(keys, values, addr):
    M, D = keys.shape
    Q = addr.shape[0]
    V = values.shape[1]
    num_blocks = M // _BM

    addr_s = (addr * _TEMPERATURE).astype(jnp.bfloat16)
    keysT = keys.astype(jnp.bfloat16).T
    vals1 = jnp.concatenate(
        [values, jnp.ones((M, 1), values.dtype)], axis=1
    ).astype(jnp.bfloat16)

    return pl.pallas_call(
        functools.partial(_flash_body, num_blocks=num_blocks),
        grid=(num_blocks,),
        in_specs=[
            pl.BlockSpec((Q, D), lambda i: (0, 0)),
            pl.BlockSpec((D, _BM), lambda i: (0, i)),
            pl.BlockSpec((_BM, V + 1), lambda i: (i, 0)),
        ],
        out_specs=pl.BlockSpec((Q, V), lambda i: (0, 0)),
        out_shape=jax.ShapeDtypeStruct((Q, V), jnp.float32),
        scratch_shapes=[
            pltpu.VMEM((Q, 1), jnp.bfloat16),
            pltpu.VMEM((Q, V + 1), jnp.float32),
        ],
        compiler_params=pltpu.CompilerParams(
            dimension_semantics=("arbitrary",),
        ),
    )(addr_s, keysT, vals1)


# speedup vs baseline: 1.4442x; 1.2541x over previous
import functools

import jax
import jax.numpy as jnp
from jax.experimental import pallas as pl
from jax.experimental.pallas import tpu as pltpu

_TEMPERATURE = 10.0
_BM = 16384


def _flash_body(addr_ref, keysT_ref, values_ref, out_ref, m_ref, acc_ref,
                *, num_blocks):
    i = pl.program_id(0)

    @pl.when(i == 0)
    def _init():
        m_ref[...] = jnp.full_like(m_ref, -jnp.inf)
        acc_ref[...] = jnp.zeros_like(acc_ref)

    s = jax.lax.dot_general(
        addr_ref[...], keysT_ref[...],
        (((1,), (0,)), ((), ())),
        preferred_element_type=jnp.float32,
    ).astype(jnp.bfloat16)
    m_prev = m_ref[...]
    m_new = jnp.maximum(m_prev, jnp.max(s, axis=1, keepdims=True))
    alpha = jnp.exp(m_prev - m_new)
    p = jnp.exp(s - m_new)
    pv = jax.lax.dot_general(
        p, values_ref[...],
        (((1,), (1,)), ((), ())),
        preferred_element_type=jnp.float32,
    )
    m_ref[...] = m_new
    acc_ref[...] = acc_ref[...] * alpha.astype(jnp.float32) + pv

    @pl.when(i == num_blocks - 1)
    def _fini():
        acc = acc_ref[...]
        out_ref[...] = acc[:, :-1] / acc[:, -1:]


@jax.jit
def kernel(keys, values, addr):
    M, D = keys.shape
    Q = addr.shape[0]
    V = values.shape[1]
    num_blocks = M // _BM

    addr_s = (addr * _TEMPERATURE).astype(jnp.bfloat16)
    keysT = keys.astype(jnp.bfloat16).T
    vals1T = jnp.concatenate(
        [values, jnp.ones((M, 1), values.dtype)], axis=1
    ).astype(jnp.bfloat16).T

    return pl.pallas_call(
        functools.partial(_flash_body, num_blocks=num_blocks),
        grid=(num_blocks,),
        in_specs=[
            pl.BlockSpec((Q, D), lambda i: (0, 0)),
            pl.BlockSpec((D, _BM), lambda i: (0, i)),
            pl.BlockSpec((V + 1, _BM), lambda i: (0, i)),
        ],
        out_specs=pl.BlockSpec((Q, V), lambda i: (0, 0)),
        out_shape=jax.ShapeDtypeStruct((Q, V), jnp.float32),
        scratch_shapes=[
            pltpu.VMEM((Q, 1), jnp.bfloat16),
            pltpu.VMEM((Q, V + 1), jnp.float32),
        ],
        compiler_params=pltpu.CompilerParams(
            dimension_semantics=("arbitrary",),
        ),
    )(addr_s, keysT, vals1T)
